# baseline (speedup 1.0000x reference)
import functools

import jax
import jax.numpy as jnp
from jax import lax
from jax.experimental import pallas as pl
from jax.experimental.pallas import tpu as pltpu
from jax.experimental.pallas import tpu_sc as plsc

N_CLS = 1000
N_CTX = 4
SEQ = 77
DIM = 512
SUF = SEQ - 1 - N_CTX
NC, NS = 2, 16
NW = NC * NS
BPW = 32
CHUNKS = (SUF + 15) // 16


def _sc_assemble(prefix2, suffix2, ctx2, label_p):
    mesh = plsc.VectorSubcoreMesh(
        core_axis_name="c", subcore_axis_name="s",
        num_cores=NC, num_subcores=NS)

    @functools.partial(
        pl.kernel,
        out_type=jax.ShapeDtypeStruct((N_CLS * SEQ, DIM), jnp.float32),
        mesh=mesh,
        compiler_params=pltpu.CompilerParams(
            use_tc_tiling_on_sc=False, needs_layout_passes=False),
        scratch_types=[
            pltpu.VMEM((BPW,), jnp.int32),
            pltpu.VMEM((BPW, DIM), jnp.float32),
            pltpu.VMEM((N_CTX, DIM), jnp.float32),
            pltpu.VMEM((CHUNKS * 16, DIM), jnp.float32),
            pltpu.VMEM((CHUNKS * 16, DIM), jnp.float32),
            pltpu.SemaphoreType.DMA,
            pltpu.SemaphoreType.DMA,
            pltpu.SemaphoreType.DMA,
        ],
    )
    def k(prefix_h, suffix_h, ctx_h, label_h, out_h,
          lbl_v, pre_v, ctx_v, buf_a, buf_b, sem_a, sem_b, sem_p):
        wid = lax.axis_index("s") * NC + lax.axis_index("c")
        base = wid * BPW
        n = jnp.minimum(BPW, N_CLS - base)
        lane = lax.iota(jnp.int32, 16)

        pltpu.sync_copy(label_h.at[pl.ds(base, BPW)], lbl_v)
        cp0 = pltpu.async_copy(
            prefix_h.at[lbl_v[pl.ds(0, 16)]], pre_v.at[pl.ds(0, 16)], sem_p)
        cp1 = pltpu.async_copy(
            prefix_h.at[lbl_v[pl.ds(16, 16)]], pre_v.at[pl.ds(16, 16)], sem_p)
        pltpu.sync_copy(ctx_h, ctx_v)
        cp0.wait()
        cp1.wait()

        def chunk_copy(cc, buf, sem, t):
            lv = plsc.load_gather(lbl_v, [jnp.full((16,), cc, jnp.int32)])
            roff = jnp.minimum(16 * t + lane, SUF - 1)
            return pltpu.make_async_copy(
                suffix_h.at[lv * SUF + roff], buf.at[pl.ds(16 * t, 16)], sem)

        def issue(cc, buf, sem):
            for t in range(CHUNKS):
                chunk_copy(cc, buf, sem, t).start()

        def drain(cc, buf, sem):
            for t in range(CHUNKS):
                chunk_copy(cc, buf, sem, t).wait()

        def write(cc, buf):
            orow = (base + cc) * SEQ
            pltpu.sync_copy(pre_v.at[pl.ds(cc, 1)], out_h.at[pl.ds(orow, 1)])
            pltpu.sync_copy(ctx_v, out_h.at[pl.ds(orow + 1, N_CTX)])
            pltpu.sync_copy(buf.at[pl.ds(0, SUF)],
                            out_h.at[pl.ds(orow + 1 + N_CTX, SUF)])

        def body(t, carry):
            c0 = 2 * t

            @pl.when(t == 0)
            def _():
                issue(c0, buf_a, sem_a)

            drain(c0, buf_a, sem_a)
            issue(c0 + 1, buf_b, sem_b)
            write(c0, buf_a)
            drain(c0 + 1, buf_b, sem_b)

            @pl.when(c0 + 2 < n)
            def _():
                issue(c0 + 2, buf_a, sem_a)

            write(c0 + 1, buf_b)
            return carry

        lax.fori_loop(0, n // 2, body, jnp.int32(0))

    return k(prefix2, suffix2, ctx2, label_p)


def kernel(token_prefix, token_suffix, ctx_vectors, label):
    prefix2 = token_prefix.reshape(N_CLS, DIM)
    suffix2 = token_suffix.reshape(N_CLS * SUF, DIM)
    label_p = jnp.pad(label.astype(jnp.int32), (0, NW * BPW - N_CLS))
    out = _sc_assemble(prefix2, suffix2, ctx_vectors, label_p)
    return out.reshape(N_CLS, SEQ, DIM)


# speedup vs baseline: 1.0376x; 1.0376x over previous
import functools

import jax
import jax.numpy as jnp
from jax import lax
from jax.experimental import pallas as pl
from jax.experimental.pallas import tpu as pltpu
from jax.experimental.pallas import tpu_sc as plsc

N_CLS = 1000
N_CTX = 4
SEQ = 77
DIM = 512
SUF = SEQ - 1 - N_CTX
NC, NS = 2, 16
NW = NC * NS
BPW = 32
OFFS = (0, 16, 32, 48, SUF - 16)


def _sc_assemble(prefix2, suffix2, ctx2, label_p):
    mesh = plsc.VectorSubcoreMesh(
        core_axis_name="c", subcore_axis_name="s",
        num_cores=NC, num_subcores=NS)

    @functools.partial(
        pl.kernel,
        out_type=jax.ShapeDtypeStruct((N_CLS * SEQ, DIM), jnp.float32),
        mesh=mesh,
        compiler_params=pltpu.CompilerParams(
            use_tc_tiling_on_sc=False, needs_layout_passes=False),
        scratch_types=[
            pltpu.VMEM((BPW,), jnp.int32),
            pltpu.VMEM((BPW, DIM), jnp.float32),
            pltpu.VMEM((SUF,), jnp.int32),
            pltpu.VMEM((SUF,), jnp.int32),
            pltpu.VMEM((SEQ, DIM), jnp.float32),
            pltpu.VMEM((SEQ, DIM), jnp.float32),
            pltpu.SemaphoreType.DMA,
            pltpu.SemaphoreType.DMA,
            pltpu.SemaphoreType.DMA,
        ],
    )
    def k(prefix_h, suffix_h, ctx_h, label_h, out_h,
          lbl_v, pre_v, idx_a, idx_b, buf_a, buf_b, sem_a, sem_b, sem_p):
        wid = lax.axis_index("s") * NC + lax.axis_index("c")
        base = wid * BPW
        n = jnp.minimum(BPW, N_CLS - base)
        lane = lax.iota(jnp.int32, 16)

        pltpu.sync_copy(label_h.at[pl.ds(base, BPW)], lbl_v)
        cp0 = pltpu.async_copy(
            prefix_h.at[lbl_v[pl.ds(0, 16)]], pre_v.at[pl.ds(0, 16)], sem_p)
        cp1 = pltpu.async_copy(
            prefix_h.at[lbl_v[pl.ds(16, 16)]], pre_v.at[pl.ds(16, 16)], sem_p)
        pltpu.sync_copy(ctx_h, buf_a.at[pl.ds(1, N_CTX)])
        pltpu.sync_copy(ctx_h, buf_b.at[pl.ds(1, N_CTX)])
        cp0.wait()
        cp1.wait()

        def gather_copy(idx, buf, sem):
            return pltpu.make_async_copy(
                suffix_h.at[idx], buf.at[pl.ds(1 + N_CTX, SUF)], sem)

        def issue(cc, idx, buf, sem):
            lv = plsc.load_gather(lbl_v, [jnp.full((16,), cc, jnp.int32)])
            for off in OFFS:
                idx[pl.ds(off, 16)] = lv * SUF + (off + lane)
            gather_copy(idx, buf, sem).start()

        def write(cc, idx, buf, sem):
            gather_copy(idx, buf, sem).wait()
            orow = (base + cc) * SEQ
            ccv = jnp.full((16,), cc, jnp.int32)
            for i in range(DIM // 16):
                buf[0, pl.ds(16 * i, 16)] = plsc.load_gather(
                    pre_v, [ccv, 16 * i + lane])
            pltpu.sync_copy(buf, out_h.at[pl.ds(orow, SEQ)])

        def body(t, carry):
            c0 = 2 * t

            @pl.when(t == 0)
            def _():
                issue(c0, idx_a, buf_a, sem_a)

            issue(c0 + 1, idx_b, buf_b, sem_b)
            write(c0, idx_a, buf_a, sem_a)

            @pl.when(c0 + 2 < n)
            def _():
                issue(c0 + 2, idx_a, buf_a, sem_a)

            write(c0 + 1, idx_b, buf_b, sem_b)
            return carry

        lax.fori_loop(0, n // 2, body, jnp.int32(0))

    return k(prefix2, suffix2, ctx2, label_p)


def kernel(token_prefix, token_suffix, ctx_vectors, label):
    prefix2 = token_prefix.reshape(N_CLS, DIM)
    suffix2 = token_suffix.reshape(N_CLS * SUF, DIM)
    label_p = jnp.pad(label.astype(jnp.int32), (0, NW * BPW - N_CLS))
    out = _sc_assemble(prefix2, suffix2, ctx_vectors, label_p)
    return out.reshape(N_CLS, SEQ, DIM)
